# baseline (speedup 1.0000x reference)
import functools

import jax
import jax.numpy as jnp
from jax.experimental import pallas as pl
from jax.experimental.pallas import tpu as pltpu

_K = 1024
_D = 64
_T = 512
_COMMIT = 0.25


def _vq_kernel(x_ref, x2_ref, e_ref, e2_ref, q_ref, idx_ref, loss_ref):
    i = pl.program_id(0)
    n_steps = pl.num_programs(0)

    x = x_ref[...]
    e = e_ref[...]
    x2 = x2_ref[...]
    e2 = e2_ref[...]

    m = jax.lax.dot_general(
        x, e, (((1,), (1,)), ((), ())),
        precision=jax.lax.Precision.DEFAULT,
        preferred_element_type=jnp.float32)
    dist = (x2 + e2) - 2.0 * m

    iota = jax.lax.broadcasted_iota(jnp.int32, (_T, _K), 1)
    dmin = jnp.min(dist, axis=1, keepdims=True)
    idx = jnp.min(jnp.where(dist == dmin, iota, _K), axis=1)

    onehot = (idx[:, None] == iota).astype(jnp.float32)
    q = jax.lax.dot_general(
        onehot, e, (((1,), (0,)), ((), ())),
        precision=jax.lax.Precision.DEFAULT,
        preferred_element_type=jnp.float32)

    diff = q - x
    q_ref[...] = x + diff
    idx_ref[...] = idx[None, None, :]

    part = jnp.sum(diff * diff)[None, None]

    @pl.when(i == 0)
    def _init():
        loss_ref[...] = part

    @pl.when(i > 0)
    def _acc():
        loss_ref[...] += part

    @pl.when(i == n_steps - 1)
    def _fin():
        mean = loss_ref[...] / jnp.float32(n_steps * _T * _D)
        loss_ref[...] = mean + jnp.float32(_COMMIT) * mean


@functools.partial(jax.jit, static_argnames=())
def kernel(inputs, embedding):
    n = inputs.shape[0] * inputs.shape[1]
    flat = inputs.reshape(n, _D)
    x2 = jnp.sum(flat ** 2, axis=1, keepdims=True)
    e2 = jnp.sum(embedding ** 2, axis=1)[None, :]

    grid = (n // _T,)
    q, idx, loss = pl.pallas_call(
        _vq_kernel,
        grid=grid,
        in_specs=[
            pl.BlockSpec((_T, _D), lambda i: (i, 0)),
            pl.BlockSpec((_T, 1), lambda i: (i, 0)),
            pl.BlockSpec((_K, _D), lambda i: (0, 0)),
            pl.BlockSpec((1, _K), lambda i: (0, 0)),
        ],
        out_specs=[
            pl.BlockSpec((_T, _D), lambda i: (i, 0)),
            pl.BlockSpec((1, 1, _T), lambda i: (i, 0, 0)),
            pl.BlockSpec((1, 1), lambda i: (0, 0)),
        ],
        out_shape=[
            jax.ShapeDtypeStruct((n, _D), jnp.float32),
            jax.ShapeDtypeStruct((grid[0], 1, _T), jnp.int32),
            jax.ShapeDtypeStruct((1, 1), jnp.float32),
        ],
    )(flat, x2, embedding, e2)

    quantized_st = q.reshape(inputs.shape)
    encoding_indices = idx.reshape(n, 1)
    return (quantized_st, loss[0, 0], encoding_indices)


# speedup vs baseline: 1.0572x; 1.0572x over previous
import functools

import jax
import jax.numpy as jnp
from jax.experimental import pallas as pl
from jax.experimental.pallas import tpu as pltpu

_K = 1024
_D = 64
_T = 1024
_COMMIT = 0.25


def _vq_kernel(x_ref, x2_ref, e_ref, ne2_ref, e2_ref, q_ref, idx_ref,
               loss_ref):
    i = pl.program_id(0)
    n_steps = pl.num_programs(0)

    x = x_ref[...]
    ne2 = ne2_ref[...]
    x2 = x2_ref[...]
    e2 = e2_ref[...]

    m2 = jax.lax.dot_general(
        x, ne2, (((1,), (1,)), ((), ())),
        precision=jax.lax.Precision.DEFAULT,
        preferred_element_type=jnp.float32)
    dist = (x2 + e2) + m2

    iota = jax.lax.broadcasted_iota(jnp.int32, (_T, _K), 1)
    dmin = jnp.min(dist, axis=1, keepdims=True)
    idx = jnp.min(jnp.where(dist == dmin, iota, _K), axis=1)

    onehot = (idx[:, None] == iota).astype(jnp.float32)
    q = jax.lax.dot_general(
        onehot, e_ref[...], (((1,), (0,)), ((), ())),
        precision=jax.lax.Precision.DEFAULT,
        preferred_element_type=jnp.float32)

    diff = q - x
    q_ref[...] = x + diff
    idx_ref[...] = idx[None, None, :]

    part = jnp.sum(diff * diff)[None, None]

    @pl.when(i == 0)
    def _init():
        loss_ref[...] = part

    @pl.when(i > 0)
    def _acc():
        loss_ref[...] += part

    @pl.when(i == n_steps - 1)
    def _fin():
        mean = loss_ref[...] / jnp.float32(n_steps * _T * _D)
        loss_ref[...] = mean + jnp.float32(_COMMIT) * mean


@functools.partial(jax.jit, static_argnames=())
def kernel(inputs, embedding):
    n = inputs.shape[0] * inputs.shape[1]
    flat = inputs.reshape(n, _D)
    x2 = jnp.sum(flat ** 2, axis=1, keepdims=True)
    e2 = jnp.sum(embedding ** 2, axis=1)[None, :]
    ne2 = -2.0 * embedding

    grid = (n // _T,)
    q, idx, loss = pl.pallas_call(
        _vq_kernel,
        grid=grid,
        in_specs=[
            pl.BlockSpec((_T, _D), lambda i: (i, 0)),
            pl.BlockSpec((_T, 1), lambda i: (i, 0)),
            pl.BlockSpec((_K, _D), lambda i: (0, 0)),
            pl.BlockSpec((_K, _D), lambda i: (0, 0)),
            pl.BlockSpec((1, _K), lambda i: (0, 0)),
        ],
        out_specs=[
            pl.BlockSpec((_T, _D), lambda i: (i, 0)),
            pl.BlockSpec((1, 1, _T), lambda i: (i, 0, 0)),
            pl.BlockSpec((1, 1), lambda i: (0, 0)),
        ],
        out_shape=[
            jax.ShapeDtypeStruct((n, _D), jnp.float32),
            jax.ShapeDtypeStruct((grid[0], 1, _T), jnp.int32),
            jax.ShapeDtypeStruct((1, 1), jnp.float32),
        ],
    )(flat, x2, embedding, ne2, e2)

    quantized_st = q.reshape(inputs.shape)
    encoding_indices = idx.reshape(n, 1)
    return (quantized_st, loss[0, 0], encoding_indices)
